# baseline (speedup 1.0000x reference)
import functools

import jax
import jax.numpy as jnp
from jax import lax
from jax.experimental import pallas as pl
from jax.experimental.pallas import tpu as pltpu
from jax.experimental.pallas import tpu_sc as plsc

_N = 10000
_NUM_WORKERS = 32
_PER_WORKER = 320
_NPAD = _NUM_WORKERS * _PER_WORKER
_VECS = _PER_WORKER // 16

_ROTATIONS = ((13, 15, 26, 6), (17, 29, 16, 24))
_KEY_LO = 42


def _rotl(v, r):
    return lax.shift_left(v, jnp.uint32(r)) | lax.shift_right_logical(
        v, jnp.uint32(32 - r))


def _keep_vec(x1):
    k0 = jnp.uint32(0)
    k1 = jnp.uint32(_KEY_LO)
    ks = (k0, k1, k0 ^ k1 ^ jnp.uint32(0x1BD11BDA))
    x0 = jnp.zeros((16,), jnp.uint32) + ks[0]
    x1 = x1 + ks[1]
    for i in range(5):
        for r in _ROTATIONS[i % 2]:
            x0 = x0 + x1
            x1 = _rotl(x1, r)
            x1 = x1 ^ x0
        x0 = x0 + ks[(i + 1) % 3]
        x1 = x1 + ks[(i + 2) % 3] + jnp.uint32(i + 1)
    bits = x0 ^ x1
    keep = lax.shift_right_logical(bits, jnp.uint32(9)) > jnp.uint32(419430)
    return jnp.where(keep, jnp.int32(1), jnp.int32(0))


@functools.lru_cache(maxsize=1)
def _build_mask_kernel():
    @functools.partial(
        pl.kernel,
        mesh=plsc.VectorSubcoreMesh(core_axis_name="c", subcore_axis_name="s"),
        out_type=jax.ShapeDtypeStruct((_NPAD,), jnp.int32),
        scratch_types=[pltpu.VMEM((_PER_WORKER,), jnp.int32)],
    )
    def _mask_kernel(out_hbm, buf):
        wid = lax.axis_index("s") * 2 + lax.axis_index("c")
        base = wid * _PER_WORKER
        lane = lax.iota(jnp.uint32, 16)

        def body(j, carry):
            cnt = lane + lax.convert_element_type(base + j * 16, jnp.uint32)
            buf[pl.ds(j * 16, 16)] = _keep_vec(cnt)
            return carry

        lax.fori_loop(0, _VECS, body, 0)
        pltpu.sync_copy(buf, out_hbm.at[pl.ds(base, _PER_WORKER)])

    return _mask_kernel


def kernel(x, y, edge_index):
    padded = _build_mask_kernel()()
    mask = padded[:_N] != 0
    return (x, edge_index, y, mask, mask)
